# baseline (speedup 1.0000x reference)
import functools

import jax
import jax.numpy as jnp
import numpy as np
from jax import lax
from jax.experimental import pallas as pl
from jax.experimental.pallas import tpu as pltpu
from jax.experimental.pallas import tpu_sc as plsc

_B = 16384
_D = 128
_NC = 2
_NS = 16
_NW = _NC * _NS
_RPW = _B // _NW
_CR = 384
_CHUNKS = (128, 384)
_NCHUNK = _RPW // _CR
_L = 16


def _sc_body(
    s_hbm, w_hbm, oh_hbm, lp_hbm,
    s_a, s_b, w_v, d_v, lp_v, oh_v, sem_a, sem_b,
):
    wid = lax.axis_index("s") * _NC + lax.axis_index("c")
    base = wid * _RPW
    bufs = (s_a, s_b)
    sems = (sem_a, sem_b)

    copies = [None, None]
    copies[0] = pltpu.async_copy(
        s_hbm.at[pl.ds(base, _CHUNKS[0]), :],
        s_a.at[pl.ds(0, _CHUNKS[0]), :],
        sem_a,
    )
    pltpu.sync_copy(w_hbm, w_v)

    lanes = lax.iota(jnp.int32, 16)

    def _bf16_round(x):
        c = x * 65537.0
        return c - (c - x)

    wd = []
    for j in range(8):
        pair_idx = (lanes + (16 * j)) * 2
        w0 = plsc.load_gather(w_v, [pair_idx])
        w1 = plsc.load_gather(w_v, [pair_idx + 1])
        wd.append(_bf16_round(w1) - _bf16_round(w0))

    lane15 = lanes == (_L - 1)
    zero_idx = lanes * 0

    row_off = 0
    for c, nrows in enumerate(_CHUNKS):
        if c + 1 < len(_CHUNKS):
            copies[(c + 1) % 2] = pltpu.async_copy(
                s_hbm.at[pl.ds(base + row_off + nrows, _CHUNKS[c + 1]), :],
                bufs[(c + 1) % 2].at[pl.ds(0, _CHUNKS[c + 1]), :],
                sems[(c + 1) % 2],
            )
        copies[c % 2].wait()
        buf = bufs[c % 2]

        @plsc.parallel_loop(0, nrows, step=1, unroll=4)
        def _row(r, _buf=buf, _off=row_off):
            acc = _bf16_round(_buf[r, pl.ds(0, _L)]) * wd[0]
            for j in range(1, 8):
                acc = acc + _bf16_round(_buf[r, pl.ds(16 * j, _L)]) * wd[j]
            plsc.store_scatter(
                d_v, [zero_idx + (_off + r)], jnp.cumsum(acc), mask=lane15
            )

        row_off += nrows

    col_is_one = (lanes & 1) == 1
    pair_lo = lanes >> 1
    pair_hi = pair_lo + 8

    @plsc.parallel_loop(0, _RPW, step=_L)
    def _tail(r0):
        d = d_v[pl.ds(r0, _L)]
        y = jnp.exp(-jnp.abs(d))
        r = y / (2.0 + y)
        r2 = r * r
        poly = 1.0 + r2 * (
            (1.0 / 3.0)
            + r2 * ((1.0 / 5.0) + r2 * ((1.0 / 7.0) + r2 * (1.0 / 9.0)))
        )
        lp_v[pl.ds(r0, _L)] = (-2.0) * r * poly

        dlo = jnp.take(d, pair_lo, mode="wrap")
        dhi = jnp.take(d, pair_hi, mode="wrap")
        oh_v[pl.ds(2 * r0, _L)] = jnp.where(col_is_one == (dlo > 0.0), 1.0, 0.0)
        oh_v[pl.ds(2 * r0 + _L, _L)] = jnp.where(col_is_one == (dhi > 0.0), 1.0, 0.0)

    pltpu.sync_copy(oh_v, oh_hbm.at[pl.ds(2 * base, 2 * _RPW)])
    pltpu.sync_copy(lp_v, lp_hbm.at[pl.ds(base, _RPW)])


@jax.jit
def _run(s, W):
    mesh = plsc.VectorSubcoreMesh(core_axis_name="c", subcore_axis_name="s")
    f = pl.kernel(
        _sc_body,
        mesh=mesh,
        compiler_params=pltpu.CompilerParams(
            needs_layout_passes=False,
            skip_device_barrier=True,
        ),
        out_type=(
            jax.ShapeDtypeStruct((2 * _B,), jnp.float32),
            jax.ShapeDtypeStruct((_B,), jnp.float32),
        ),
        scratch_types=[
            pltpu.VMEM((_CR, _D), jnp.float32),
            pltpu.VMEM((_CR, _D), jnp.float32),
            pltpu.VMEM((_D * 2,), jnp.float32),
            pltpu.VMEM((_RPW,), jnp.float32),
            pltpu.VMEM((_RPW,), jnp.float32),
            pltpu.VMEM((2 * _RPW,), jnp.float32),
            pltpu.SemaphoreType.DMA,
            pltpu.SemaphoreType.DMA,
        ],
    )
    oh_flat, lp = f(s, W.reshape(-1))
    return oh_flat.reshape(_B, 2), lp


def kernel(s, W):
    return _run(s, W)


# speedup vs baseline: 1.0133x; 1.0133x over previous
import functools

import jax
import jax.numpy as jnp
import numpy as np
from jax import lax
from jax.experimental import pallas as pl
from jax.experimental.pallas import tpu as pltpu
from jax.experimental.pallas import tpu_sc as plsc

_B = 16384
_D = 128
_NC = 2
_NS = 16
_NW = _NC * _NS
_RPW = _B // _NW
_CR = 256
_CHUNKS = (256, 256)
_NCHUNK = _RPW // _CR
_L = 16


def _sc_body(
    s_hbm, w_hbm, oh_hbm, lp_hbm,
    s_a, s_b, w_v, d_v, lp_v, oh_v, sem_a, sem_b,
):
    wid = lax.axis_index("s") * _NC + lax.axis_index("c")
    base = wid * _RPW
    bufs = (s_a, s_b)
    sems = (sem_a, sem_b)

    copies = [None, None]
    copies[0] = pltpu.async_copy(
        s_hbm.at[pl.ds(base, _CHUNKS[0]), :],
        s_a.at[pl.ds(0, _CHUNKS[0]), :],
        sem_a,
    )
    pltpu.sync_copy(w_hbm, w_v)

    lanes = lax.iota(jnp.int32, 16)

    def _bf16_round(x):
        c = x * 65537.0
        return c - (c - x)

    wd = []
    for j in range(8):
        pair_idx = (lanes + (16 * j)) * 2
        w0 = plsc.load_gather(w_v, [pair_idx])
        w1 = plsc.load_gather(w_v, [pair_idx + 1])
        wd.append(_bf16_round(w1) - _bf16_round(w0))

    lane15 = lanes == (_L - 1)
    zero_idx = lanes * 0

    row_off = 0
    for c, nrows in enumerate(_CHUNKS):
        if c + 1 < len(_CHUNKS):
            copies[(c + 1) % 2] = pltpu.async_copy(
                s_hbm.at[pl.ds(base + row_off + nrows, _CHUNKS[c + 1]), :],
                bufs[(c + 1) % 2].at[pl.ds(0, _CHUNKS[c + 1]), :],
                sems[(c + 1) % 2],
            )
        copies[c % 2].wait()
        buf = bufs[c % 2]

        @plsc.parallel_loop(0, nrows, step=1, unroll=4)
        def _row(r, _buf=buf, _off=row_off):
            acc = _bf16_round(_buf[r, pl.ds(0, _L)]) * wd[0]
            for j in range(1, 8):
                acc = acc + _bf16_round(_buf[r, pl.ds(16 * j, _L)]) * wd[j]
            plsc.store_scatter(
                d_v, [zero_idx + (_off + r)], jnp.cumsum(acc), mask=lane15
            )

        row_off += nrows

    col_is_one = (lanes & 1) == 1
    pair_lo = lanes >> 1
    pair_hi = pair_lo + 8

    @plsc.parallel_loop(0, _RPW, step=_L)
    def _tail(r0):
        d = d_v[pl.ds(r0, _L)]
        y = jnp.exp(-jnp.abs(d))
        r = y / (2.0 + y)
        r2 = r * r
        poly = 1.0 + r2 * (
            (1.0 / 3.0)
            + r2 * ((1.0 / 5.0) + r2 * ((1.0 / 7.0) + r2 * (1.0 / 9.0)))
        )
        lp_v[pl.ds(r0, _L)] = (-2.0) * r * poly

        dlo = jnp.take(d, pair_lo, mode="wrap")
        dhi = jnp.take(d, pair_hi, mode="wrap")
        oh_v[pl.ds(2 * r0, _L)] = jnp.where(col_is_one == (dlo > 0.0), 1.0, 0.0)
        oh_v[pl.ds(2 * r0 + _L, _L)] = jnp.where(col_is_one == (dhi > 0.0), 1.0, 0.0)

    pltpu.sync_copy(oh_v, oh_hbm.at[pl.ds(2 * base, 2 * _RPW)])
    pltpu.sync_copy(lp_v, lp_hbm.at[pl.ds(base, _RPW)])


@jax.jit
def _run(s, W):
    mesh = plsc.VectorSubcoreMesh(core_axis_name="c", subcore_axis_name="s")
    f = pl.kernel(
        _sc_body,
        mesh=mesh,
        compiler_params=pltpu.CompilerParams(
            needs_layout_passes=False,
            skip_device_barrier=True,
        ),
        out_type=(
            jax.ShapeDtypeStruct((2 * _B,), jnp.float32),
            jax.ShapeDtypeStruct((_B,), jnp.float32),
        ),
        scratch_types=[
            pltpu.VMEM((_CR, _D), jnp.float32),
            pltpu.VMEM((_CR, _D), jnp.float32),
            pltpu.VMEM((_D * 2,), jnp.float32),
            pltpu.VMEM((_RPW,), jnp.float32),
            pltpu.VMEM((_RPW,), jnp.float32),
            pltpu.VMEM((2 * _RPW,), jnp.float32),
            pltpu.SemaphoreType.DMA,
            pltpu.SemaphoreType.DMA,
        ],
    )
    oh_flat, lp = f(s, W.reshape(-1))
    return oh_flat.reshape(_B, 2), lp


def kernel(s, W):
    return _run(s, W)


# speedup vs baseline: 1.0208x; 1.0074x over previous
import functools

import jax
import jax.numpy as jnp
import numpy as np
from jax import lax
from jax.experimental import pallas as pl
from jax.experimental.pallas import tpu as pltpu
from jax.experimental.pallas import tpu_sc as plsc

_B = 16384
_D = 128
_NC = 2
_NS = 16
_NW = _NC * _NS
_RPW = _B // _NW
_CR = 256
_CHUNKS = (256, 256)
_NCHUNK = _RPW // _CR
_L = 16


def _sc_body(
    s_hbm, w_hbm, oh_hbm, lp_hbm,
    s_a, s_b, w_v, d_v, lp_v, oh_v, sem_a, sem_b,
):
    wid = lax.axis_index("s") * _NC + lax.axis_index("c")
    base = wid * _RPW
    bufs = (s_a, s_b)
    sems = (sem_a, sem_b)

    copies = [None, None]
    copies[0] = pltpu.async_copy(
        s_hbm.at[pl.ds(base, _CHUNKS[0]), :],
        s_a.at[pl.ds(0, _CHUNKS[0]), :],
        sem_a,
    )
    pltpu.sync_copy(w_hbm, w_v)

    lanes = lax.iota(jnp.int32, 16)

    def _bf16_round(x):
        c = x * 65537.0
        return c - (c - x)

    wd = []
    for j in range(8):
        pair_idx = (lanes + (16 * j)) * 2
        w0 = plsc.load_gather(w_v, [pair_idx])
        w1 = plsc.load_gather(w_v, [pair_idx + 1])
        wd.append(_bf16_round(w1) - _bf16_round(w0))

    lane15 = lanes == (_L - 1)
    zero_idx = lanes * 0

    row_off = 0
    for c, nrows in enumerate(_CHUNKS):
        if c + 1 < len(_CHUNKS):
            copies[(c + 1) % 2] = pltpu.async_copy(
                s_hbm.at[pl.ds(base + row_off + nrows, _CHUNKS[c + 1]), :],
                bufs[(c + 1) % 2].at[pl.ds(0, _CHUNKS[c + 1]), :],
                sems[(c + 1) % 2],
            )
        copies[c % 2].wait()
        buf = bufs[c % 2]

        @plsc.parallel_loop(0, nrows, step=1, unroll=2)
        def _row(r, _buf=buf, _off=row_off):
            acc = _bf16_round(_buf[r, pl.ds(0, _L)]) * wd[0]
            for j in range(1, 8):
                acc = acc + _bf16_round(_buf[r, pl.ds(16 * j, _L)]) * wd[j]
            plsc.store_scatter(
                d_v, [zero_idx + (_off + r)], jnp.cumsum(acc), mask=lane15
            )

        row_off += nrows

    col_is_one = (lanes & 1) == 1
    pair_lo = lanes >> 1
    pair_hi = pair_lo + 8

    @plsc.parallel_loop(0, _RPW, step=_L)
    def _tail(r0):
        d = d_v[pl.ds(r0, _L)]
        y = jnp.exp(-jnp.abs(d))
        r = y / (2.0 + y)
        r2 = r * r
        poly = 1.0 + r2 * (
            (1.0 / 3.0)
            + r2 * ((1.0 / 5.0) + r2 * ((1.0 / 7.0) + r2 * (1.0 / 9.0)))
        )
        lp_v[pl.ds(r0, _L)] = (-2.0) * r * poly

        dlo = jnp.take(d, pair_lo, mode="wrap")
        dhi = jnp.take(d, pair_hi, mode="wrap")
        oh_v[pl.ds(2 * r0, _L)] = jnp.where(col_is_one == (dlo > 0.0), 1.0, 0.0)
        oh_v[pl.ds(2 * r0 + _L, _L)] = jnp.where(col_is_one == (dhi > 0.0), 1.0, 0.0)

    pltpu.sync_copy(oh_v, oh_hbm.at[pl.ds(2 * base, 2 * _RPW)])
    pltpu.sync_copy(lp_v, lp_hbm.at[pl.ds(base, _RPW)])


@jax.jit
def _run(s, W):
    mesh = plsc.VectorSubcoreMesh(core_axis_name="c", subcore_axis_name="s")
    f = pl.kernel(
        _sc_body,
        mesh=mesh,
        compiler_params=pltpu.CompilerParams(
            needs_layout_passes=False,
            skip_device_barrier=True,
        ),
        out_type=(
            jax.ShapeDtypeStruct((2 * _B,), jnp.float32),
            jax.ShapeDtypeStruct((_B,), jnp.float32),
        ),
        scratch_types=[
            pltpu.VMEM((_CR, _D), jnp.float32),
            pltpu.VMEM((_CR, _D), jnp.float32),
            pltpu.VMEM((_D * 2,), jnp.float32),
            pltpu.VMEM((_RPW,), jnp.float32),
            pltpu.VMEM((_RPW,), jnp.float32),
            pltpu.VMEM((2 * _RPW,), jnp.float32),
            pltpu.SemaphoreType.DMA,
            pltpu.SemaphoreType.DMA,
        ],
    )
    oh_flat, lp = f(s, W.reshape(-1))
    return oh_flat.reshape(_B, 2), lp


def kernel(s, W):
    return _run(s, W)
